# baseline (speedup 1.0000x reference)
import functools

import jax
import jax.numpy as jnp
from jax import lax
from jax.experimental import pallas as pl
from jax.experimental.pallas import tpu as pltpu
from jax.experimental.pallas import tpu_sc as plsc

D = 32
SUP = 128
RPS = SUP // D
LANES = 16
NC = 2
NS = 16
NW = NC * NS
CHUNK = 128


def _dwe_body(i_hbm, j_hbm, de_hbm, emb_hbm, out_hbm,
              i_v, j_v, de_v, iq_v, jq_v, u4, v4, o_v, sem, bpw):
    nchunks = bpw // CHUNK
    wid = lax.axis_index("s") * NC + lax.axis_index("c")
    base = wid * nchunks

    pltpu.sync_copy(i_hbm.at[pl.ds(base, nchunks)], i_v)
    pltpu.sync_copy(j_hbm.at[pl.ds(base, nchunks)], j_v)
    pltpu.sync_copy(de_hbm.at[pl.ds(base, nchunks)], de_v)

    def to_super(t, _):
        k = t // (CHUNK // LANES)
        o = (t % (CHUNK // LANES)) * LANES
        iq_v[k, pl.ds(o, LANES)] = i_v[k, pl.ds(o, LANES)] >> 2
        jq_v[k, pl.ds(o, LANES)] = j_v[k, pl.ds(o, LANES)] >> 2
        return _
    lax.fori_loop(0, nchunks * (CHUNK // LANES), to_super, 0)

    def fire(k):
        b = k % 2
        cu = pltpu.async_copy(emb_hbm.at[iq_v.at[k]], u4.at[b], sem)
        cv = pltpu.async_copy(emb_hbm.at[jq_v.at[k]], v4.at[b], sem)
        return cu, cv

    lane_iota = lax.broadcasted_iota(jnp.int32, (LANES,), 0)

    def compute_chunk(k):
        b = jnp.full((LANES,), k % 2, jnp.int32)

        def group(g, _):
            rows = lane_iota + g * LANES
            iv = i_v[k, pl.ds(g * LANES, LANES)]
            jv = j_v[k, pl.ds(g * LANES, LANES)]
            isub = (iv & (RPS - 1)) * D
            jsub = (jv & (RPS - 1)) * D
            acc = jnp.zeros((LANES,), jnp.float32)
            for d in range(D):
                ud = plsc.load_gather(u4, [b, rows, isub + d])
                vd = plsc.load_gather(v4, [b, rows, jsub + d])
                acc = acc + ud * vd
            dev = de_v[k, pl.ds(g * LANES, LANES)]
            x = dev * acc
            s = 1.0 / (1.0 + jnp.exp(-x))
            o_v[pl.ds(k * CHUNK + g * LANES, LANES)] = -s
            return _

        lax.fori_loop(0, CHUNK // LANES, group, 0)

    pending = fire(0)
    for k in range(nchunks):
        nxt = fire(k + 1) if k + 1 < nchunks else None
        for c in pending:
            c.wait()
        compute_chunk(k)
        pending = nxt

    pltpu.sync_copy(o_v, out_hbm.at[pl.ds(wid * bpw, bpw)])


def kernel(pair, emb):
    B = pair.shape[0]
    V = emb.shape[0]
    bpw = B // NW
    i = pair[:, 0].astype(jnp.int32).reshape(B // CHUNK, CHUNK)
    j = pair[:, 1].astype(jnp.int32).reshape(B // CHUNK, CHUNK)
    de = pair[:, 2].astype(jnp.float32).reshape(B // CHUNK, CHUNK)
    emb4 = emb.reshape(V // RPS, SUP)

    mesh = plsc.VectorSubcoreMesh(core_axis_name="c", subcore_axis_name="s")
    run = pl.kernel(
        functools.partial(_dwe_body, bpw=bpw),
        out_type=jax.ShapeDtypeStruct((B,), jnp.float32),
        mesh=mesh,
        compiler_params=pltpu.CompilerParams(needs_layout_passes=False),
        scratch_types=[
            pltpu.VMEM((bpw // CHUNK, CHUNK), jnp.int32),
            pltpu.VMEM((bpw // CHUNK, CHUNK), jnp.int32),
            pltpu.VMEM((bpw // CHUNK, CHUNK), jnp.float32),
            pltpu.VMEM((bpw // CHUNK, CHUNK), jnp.int32),
            pltpu.VMEM((bpw // CHUNK, CHUNK), jnp.int32),
            pltpu.VMEM((2, CHUNK, SUP), jnp.float32),
            pltpu.VMEM((2, CHUNK, SUP), jnp.float32),
            pltpu.VMEM((bpw,), jnp.float32),
            pltpu.SemaphoreType.DMA,
        ],
    )
    out = run(i, j, de, emb4)
    return out.reshape(B, 1)
